# baseline (speedup 1.0000x reference)
import functools

import jax
import jax.numpy as jnp
from jax import lax
from jax.experimental import pallas as pl
from jax.experimental.pallas import tpu as pltpu
from jax.experimental.pallas import tpu_sc as plsc

D = 768
E = 8
F = 3072
T = 2048
K = 2
TK = T * K
BT = 256
NB = (TK + E * (BT - 1) + BT - 1) // BT
NP = NB * BT
NW = 32
TPW = T // NW


def _router_body(x_ref, wg_ref, bg_ref,
                 de_ref, do_ref, wa_ref, wb_ref, be_ref, act_ref):
    x = x_ref[...]
    logits = jnp.dot(x, wg_ref[...], preferred_element_type=jnp.float32)
    logits = logits + bg_ref[...]

    eids = lax.broadcasted_iota(jnp.int32, (T, E), 1)
    l1 = jnp.max(logits, axis=1, keepdims=True)
    a1 = jnp.argmax(logits, axis=1).astype(jnp.int32)[:, None]
    masked = jnp.where(eids == a1, -jnp.inf, logits)
    l2 = jnp.max(masked, axis=1, keepdims=True)
    a2 = jnp.argmax(masked, axis=1).astype(jnp.int32)[:, None]
    wa = 1.0 / (1.0 + jnp.exp(l2 - l1))
    wb = 1.0 - wa

    e_slot = jnp.concatenate([a1, a2], axis=0)
    oh = (e_slot == lax.broadcasted_iota(jnp.int32, (TK, E), 1))
    oh = oh.astype(jnp.float32)

    nblk = TK // 128
    ohb = oh.reshape(nblk, 128, E)
    r = lax.broadcasted_iota(jnp.int32, (128, 128), 0)
    c = lax.broadcasted_iota(jnp.int32, (128, 128), 1)
    tri_incl = (r <= c).astype(jnp.float32)
    y = jnp.einsum('ale,lk->ake', ohb, tri_incl,
                   preferred_element_type=jnp.float32)
    totals = y[:, 127, :]
    rb = lax.broadcasted_iota(jnp.int32, (nblk, nblk), 0)
    cb = lax.broadcasted_iota(jnp.int32, (nblk, nblk), 1)
    tri_strict = (cb < rb).astype(jnp.float32)
    carry = jnp.einsum('ab,be->ae', tri_strict, totals,
                       preferred_element_type=jnp.float32)
    csum = (y + carry[:, None, :]).reshape(TK, E)

    counts = csum[TK - 1:TK, :]
    pc = jnp.ceil(counts / BT) * BT
    er = lax.broadcasted_iota(jnp.int32, (E, E), 0)
    ec = lax.broadcasted_iota(jnp.int32, (E, E), 1)
    tri_e = (er <= ec).astype(jnp.float32)
    cum_pc = jnp.einsum('ae,ef->af', pc, tri_e,
                        preferred_element_type=jnp.float32)
    po = cum_pc - pc

    rank = jnp.sum(oh * csum, axis=1, keepdims=True) - 1.0
    poe = jnp.sum(oh * po, axis=1, keepdims=True)
    dest = (poe + rank).astype(jnp.int32)
    de_ref[...] = dest[:T]
    do_ref[...] = dest[T:]
    wa_ref[...] = wa
    wb_ref[...] = wb

    s = lax.broadcasted_iota(jnp.int32, (NB, E), 0).astype(jnp.float32) * BT
    total = cum_pc[0, E - 1]
    be_raw = jnp.sum((s >= cum_pc).astype(jnp.int32), axis=1, keepdims=True)
    active = s[:, :1] < total
    eidx = lax.broadcasted_iota(jnp.int32, (1, E), 1)
    eb_last = jnp.max(jnp.where(pc > 0.0, eidx, -1), axis=1, keepdims=True)
    be = jnp.where(active, jnp.minimum(be_raw, E - 1), eb_last)
    be_ref[...] = be.astype(jnp.int32)
    act_ref[...] = active.astype(jnp.int32)


def _router_call(xf, wg, bg2):
    outs = (
        jax.ShapeDtypeStruct((T, 1), jnp.int32),
        jax.ShapeDtypeStruct((T, 1), jnp.int32),
        jax.ShapeDtypeStruct((T, 1), jnp.float32),
        jax.ShapeDtypeStruct((T, 1), jnp.float32),
        jax.ShapeDtypeStruct((NB, 1), jnp.int32),
        jax.ShapeDtypeStruct((NB, 1), jnp.int32),
    )
    return pl.pallas_call(_router_body, out_shape=outs)(xf, wg, bg2)


@functools.lru_cache(maxsize=None)
def _sc_kernels():
    mesh = plsc.VectorSubcoreMesh(core_axis_name="c", subcore_axis_name="s",
                                  num_cores=2, num_subcores=16)

    @functools.partial(
        pl.kernel,
        out_type=jax.ShapeDtypeStruct((NP, D), jnp.float32),
        mesh=mesh,
        scratch_types=[
            pltpu.VMEM((TPW, D), jnp.float32),
            pltpu.VMEM((TPW,), jnp.int32),
            pltpu.VMEM((TPW,), jnp.int32),
            pltpu.SemaphoreType.DMA,
            pltpu.SemaphoreType.DMA,
        ],
    )
    def _sc_scatter(x_hbm, de_hbm, do_hbm, out_hbm,
                    rows_v, ide_v, ido_v, s1, s2):
        wid = lax.axis_index("s") * 2 + lax.axis_index("c")
        base = wid * TPW
        pltpu.sync_copy(x_hbm.at[pl.ds(base, TPW)], rows_v)
        pltpu.sync_copy(de_hbm.at[pl.ds(base, TPW)], ide_v)
        pltpu.sync_copy(do_hbm.at[pl.ds(base, TPW)], ido_v)
        pltpu.async_copy(rows_v, out_hbm.at[ide_v], s1).wait()
        pltpu.async_copy(rows_v, out_hbm.at[ido_v], s2).wait()

    @functools.partial(
        pl.kernel,
        out_type=(jax.ShapeDtypeStruct((T, D), jnp.float32),
                  jax.ShapeDtypeStruct((T, D), jnp.float32)),
        mesh=mesh,
        scratch_types=[
            pltpu.VMEM((TPW, D), jnp.float32),
            pltpu.VMEM((TPW,), jnp.int32),
            pltpu.SemaphoreType.DMA,
        ],
    )
    def _sc_gather(ys_hbm, de_hbm, do_hbm, ye_hbm, yo_hbm,
                   rows_v, idx_v, sem):
        wid = lax.axis_index("s") * 2 + lax.axis_index("c")
        base = wid * TPW
        pltpu.sync_copy(de_hbm.at[pl.ds(base, TPW)], idx_v)
        pltpu.async_copy(ys_hbm.at[idx_v], rows_v, sem).wait()
        pltpu.sync_copy(rows_v, ye_hbm.at[pl.ds(base, TPW)])
        pltpu.sync_copy(do_hbm.at[pl.ds(base, TPW)], idx_v)
        pltpu.async_copy(ys_hbm.at[idx_v], rows_v, sem).wait()
        pltpu.sync_copy(rows_v, yo_hbm.at[pl.ds(base, TPW)])

    return _sc_scatter, _sc_gather


FS = 2
F2 = F // FS


def _ffn_body(be_ref, act_ref, x_ref, w1_ref, b1_ref, w2_ref, b2_ref, y_ref):
    b = pl.program_id(0)
    j = pl.program_id(1)
    act = act_ref[b]

    @pl.when(act == 1)
    def _compute():
        h = jnp.dot(x_ref[...], w1_ref[0], preferred_element_type=jnp.float32)
        h = h + b1_ref[0]
        h = 0.5 * h * (1.0 + lax.erf(h * 0.7071067811865476))
        part = jnp.dot(h, w2_ref[0], preferred_element_type=jnp.float32)

        @pl.when(j == 0)
        def _init():
            y_ref[...] = part + b2_ref[0]

        @pl.when(j != 0)
        def _acc():
            y_ref[...] += part

    @pl.when(jnp.logical_and(act == 0, j == 0))
    def _skip():
        y_ref[...] = jnp.zeros_like(y_ref)


def _ffn_call(be, act, x_sorted, w1, b1, w2, b2):
    grid_spec = pltpu.PrefetchScalarGridSpec(
        num_scalar_prefetch=2,
        grid=(NB, FS),
        in_specs=[
            pl.BlockSpec((BT, D), lambda b, j, be, act: (b, 0)),
            pl.BlockSpec((1, D, F2), lambda b, j, be, act: (be[b], 0, j)),
            pl.BlockSpec((1, 1, F2), lambda b, j, be, act: (be[b], 0, j)),
            pl.BlockSpec((1, F2, D), lambda b, j, be, act: (be[b], j, 0)),
            pl.BlockSpec((1, 1, D), lambda b, j, be, act: (be[b], 0, 0)),
        ],
        out_specs=pl.BlockSpec((BT, D), lambda b, j, be, act: (b, 0)),
    )
    return pl.pallas_call(
        _ffn_body,
        grid_spec=grid_spec,
        out_shape=jax.ShapeDtypeStruct((NP, D), jnp.float32),
    )(be, act, x_sorted, w1, b1.reshape(E, 1, F), w2, b2.reshape(E, 1, D))


def _combine_body(wa_ref, wb_ref, ye_ref, yo_ref, o_ref):
    o_ref[...] = wa_ref[...] * ye_ref[...] + wb_ref[...] * yo_ref[...]


def _combine_call(wa, wb, ye, yo):
    grid_spec = pl.GridSpec(
        grid=(T // BT,),
        in_specs=[
            pl.BlockSpec((BT, 1), lambda i: (i, 0)),
            pl.BlockSpec((BT, 1), lambda i: (i, 0)),
            pl.BlockSpec((BT, D), lambda i: (i, 0)),
            pl.BlockSpec((BT, D), lambda i: (i, 0)),
        ],
        out_specs=pl.BlockSpec((BT, D), lambda i: (i, 0)),
    )
    return pl.pallas_call(
        _combine_body,
        grid_spec=grid_spec,
        out_shape=jax.ShapeDtypeStruct((T, D), jnp.float32),
    )(wa, wb, ye, yo)


@jax.jit
def kernel(x, Wg, bg, W1, b1, W2, b2):
    orig_shape = x.shape
    xf = x.reshape(T, D)
    sc_scatter, sc_gather = _sc_kernels()
    de, do, wa, wb, be, act = _router_call(xf, Wg, bg.reshape(1, E))
    x_sorted = sc_scatter(xf, de.reshape(T), do.reshape(T))
    y_sorted = _ffn_call(be.reshape(NB), act.reshape(NB),
                         x_sorted, W1, b1, W2, b2)
    ye, yo = sc_gather(y_sorted, de.reshape(T), do.reshape(T))
    out = _combine_call(wa, wb, ye, yo)
    return out.reshape(orig_shape)


# speedup vs baseline: 1.4723x; 1.4723x over previous
import functools

import jax
import jax.numpy as jnp
from jax import lax
from jax.experimental import pallas as pl
from jax.experimental.pallas import tpu as pltpu
from jax.experimental.pallas import tpu_sc as plsc

D = 768
E = 8
F = 3072
T = 2048
K = 2
TK = T * K
BT = 256
NB = (TK + E * (BT - 1) + BT - 1) // BT
NP = NB * BT
NW = 32
TPW = T // NW


def _router_body(x_ref, wg_ref, bg_ref,
                 de_ref, do_ref, wa_ref, wb_ref, be_ref, act_ref,
                 fi_ref, sl_ref, nx_ref):
    x = x_ref[...]
    logits = jnp.dot(x, wg_ref[...], preferred_element_type=jnp.float32)
    logits = logits + bg_ref[...]

    eids = lax.broadcasted_iota(jnp.int32, (T, E), 1)
    l1 = jnp.max(logits, axis=1, keepdims=True)
    a1 = jnp.argmax(logits, axis=1).astype(jnp.int32)[:, None]
    masked = jnp.where(eids == a1, -jnp.inf, logits)
    l2 = jnp.max(masked, axis=1, keepdims=True)
    a2 = jnp.argmax(masked, axis=1).astype(jnp.int32)[:, None]
    wa = 1.0 / (1.0 + jnp.exp(l2 - l1))
    wb = 1.0 - wa

    e_slot = jnp.concatenate([a1, a2], axis=0)
    oh = (e_slot == lax.broadcasted_iota(jnp.int32, (TK, E), 1))
    oh = oh.astype(jnp.float32)

    nblk = TK // 128
    ohb = oh.reshape(nblk, 128, E)
    r = lax.broadcasted_iota(jnp.int32, (128, 128), 0)
    c = lax.broadcasted_iota(jnp.int32, (128, 128), 1)
    tri_incl = (r <= c).astype(jnp.float32)
    y = jnp.einsum('ale,lk->ake', ohb, tri_incl,
                   preferred_element_type=jnp.float32)
    totals = y[:, 127, :]
    rb = lax.broadcasted_iota(jnp.int32, (nblk, nblk), 0)
    cb = lax.broadcasted_iota(jnp.int32, (nblk, nblk), 1)
    tri_strict = (cb < rb).astype(jnp.float32)
    carry = jnp.einsum('ab,be->ae', tri_strict, totals,
                       preferred_element_type=jnp.float32)
    csum = (y + carry[:, None, :]).reshape(TK, E)

    counts = csum[TK - 1:TK, :]
    pc = jnp.ceil(counts / BT) * BT
    er = lax.broadcasted_iota(jnp.int32, (E, E), 0)
    ec = lax.broadcasted_iota(jnp.int32, (E, E), 1)
    tri_e = (er <= ec).astype(jnp.float32)
    cum_pc = jnp.einsum('ae,ef->af', pc, tri_e,
                        preferred_element_type=jnp.float32)
    po = cum_pc - pc

    rank = jnp.sum(oh * csum, axis=1, keepdims=True) - 1.0
    poe = jnp.sum(oh * po, axis=1, keepdims=True)
    dest = (poe + rank).astype(jnp.int32)
    de_ref[...] = dest[:T]
    do_ref[...] = dest[T:]
    wa_ref[...] = wa
    wb_ref[...] = wb

    s = lax.broadcasted_iota(jnp.int32, (NB, E), 0).astype(jnp.float32) * BT
    total = cum_pc[0, E - 1]
    be_raw = jnp.sum((s >= cum_pc).astype(jnp.int32), axis=1, keepdims=True)
    active = s[:, :1] < total
    eidx = lax.broadcasted_iota(jnp.int32, (1, E), 1)
    eb_last = jnp.max(jnp.where(pc > 0.0, eidx, -1), axis=1, keepdims=True)
    be = jnp.where(active, jnp.minimum(be_raw, E - 1), eb_last)
    be = be.astype(jnp.int32)
    be_ref[...] = be
    act_ref[...] = active.astype(jnp.int32)

    be_prev = jnp.concatenate(
        [jnp.full((1, 1), -1, jnp.int32), be[:-1]], axis=0)
    firstf = (be != be_prev).astype(jnp.float32)
    rnb = lax.broadcasted_iota(jnp.int32, (NB, NB), 0)
    cnb = lax.broadcasted_iota(jnp.int32, (NB, NB), 1)
    tri_nb = (cnb <= rnb).astype(jnp.float32)
    gidx = jnp.einsum('ab,bk->ak', tri_nb, firstf,
                      preferred_element_type=jnp.float32)
    slot = (gidx.astype(jnp.int32) - 1) % 2
    mask_next = jnp.logical_and(cnb > rnb, firstf[:, 0][None, :] > 0.0)
    pos = jnp.min(jnp.where(mask_next, cnb, 2 * NB), axis=1, keepdims=True)
    onehot_pos = (cnb == pos).astype(jnp.float32)
    nxt_f = jnp.einsum('ab,bk->ak', onehot_pos, be.astype(jnp.float32),
                       preferred_element_type=jnp.float32)
    nxt = jnp.where(pos < NB, nxt_f.astype(jnp.int32), -1)
    fi_ref[...] = firstf.astype(jnp.int32)
    sl_ref[...] = slot
    nx_ref[...] = nxt


def _router_call(xf, wg, bg2):
    outs = (
        jax.ShapeDtypeStruct((T, 1), jnp.int32),
        jax.ShapeDtypeStruct((T, 1), jnp.int32),
        jax.ShapeDtypeStruct((T, 1), jnp.float32),
        jax.ShapeDtypeStruct((T, 1), jnp.float32),
        jax.ShapeDtypeStruct((NB, 1), jnp.int32),
        jax.ShapeDtypeStruct((NB, 1), jnp.int32),
        jax.ShapeDtypeStruct((NB, 1), jnp.int32),
        jax.ShapeDtypeStruct((NB, 1), jnp.int32),
        jax.ShapeDtypeStruct((NB, 1), jnp.int32),
    )
    return pl.pallas_call(_router_body, out_shape=outs)(xf, wg, bg2)


@functools.lru_cache(maxsize=None)
def _sc_kernels():
    mesh = plsc.VectorSubcoreMesh(core_axis_name="c", subcore_axis_name="s",
                                  num_cores=2, num_subcores=16)

    @functools.partial(
        pl.kernel,
        out_type=jax.ShapeDtypeStruct((NP, D), jnp.float32),
        mesh=mesh,
        scratch_types=[
            pltpu.VMEM((TPW, D), jnp.float32),
            pltpu.VMEM((TPW,), jnp.int32),
            pltpu.VMEM((TPW,), jnp.int32),
            pltpu.SemaphoreType.DMA,
            pltpu.SemaphoreType.DMA,
        ],
    )
    def _sc_scatter(x_hbm, de_hbm, do_hbm, out_hbm,
                    rows_v, ide_v, ido_v, s1, s2):
        wid = lax.axis_index("s") * 2 + lax.axis_index("c")
        base = wid * TPW
        pltpu.sync_copy(x_hbm.at[pl.ds(base, TPW)], rows_v)
        pltpu.sync_copy(de_hbm.at[pl.ds(base, TPW)], ide_v)
        pltpu.sync_copy(do_hbm.at[pl.ds(base, TPW)], ido_v)
        pltpu.async_copy(rows_v, out_hbm.at[ide_v], s1).wait()
        pltpu.async_copy(rows_v, out_hbm.at[ido_v], s2).wait()

    @functools.partial(
        pl.kernel,
        out_type=(jax.ShapeDtypeStruct((T, D), jnp.float32),
                  jax.ShapeDtypeStruct((T, D), jnp.float32)),
        mesh=mesh,
        scratch_types=[
            pltpu.VMEM((TPW, D), jnp.float32),
            pltpu.VMEM((TPW,), jnp.int32),
            pltpu.SemaphoreType.DMA,
        ],
    )
    def _sc_gather(ys_hbm, de_hbm, do_hbm, ye_hbm, yo_hbm,
                   rows_v, idx_v, sem):
        wid = lax.axis_index("s") * 2 + lax.axis_index("c")
        base = wid * TPW
        pltpu.sync_copy(de_hbm.at[pl.ds(base, TPW)], idx_v)
        pltpu.async_copy(ys_hbm.at[idx_v], rows_v, sem).wait()
        pltpu.sync_copy(rows_v, ye_hbm.at[pl.ds(base, TPW)])
        pltpu.sync_copy(do_hbm.at[pl.ds(base, TPW)], idx_v)
        pltpu.async_copy(ys_hbm.at[idx_v], rows_v, sem).wait()
        pltpu.sync_copy(rows_v, yo_hbm.at[pl.ds(base, TPW)])

    return _sc_scatter, _sc_gather


def _ffn_body(be_ref, act_ref, fi_ref, sl_ref, nx_ref,
              x_ref, w1_hbm, b1_ref, w2_hbm, b2_ref, y_ref,
              w1b, w2b, sem):
    b = pl.program_id(0)
    s = sl_ref[b]

    @pl.when(b == 0)
    def _prime():
        pltpu.make_async_copy(w1_hbm.at[be_ref[0]], w1b.at[0],
                              sem.at[0, 0]).start()
        pltpu.make_async_copy(w2_hbm.at[be_ref[0]], w2b.at[0],
                              sem.at[0, 1]).start()

    @pl.when(jnp.logical_and(fi_ref[b] == 1, nx_ref[b] >= 0))
    def _prefetch_next():
        nx = nx_ref[b]
        pltpu.make_async_copy(w1_hbm.at[nx], w1b.at[1 - s],
                              sem.at[1 - s, 0]).start()
        pltpu.make_async_copy(w2_hbm.at[nx], w2b.at[1 - s],
                              sem.at[1 - s, 1]).start()

    @pl.when(fi_ref[b] == 1)
    def _wait_cur():
        pltpu.make_async_copy(w1_hbm.at[be_ref[b]], w1b.at[s],
                              sem.at[s, 0]).wait()
        pltpu.make_async_copy(w2_hbm.at[be_ref[b]], w2b.at[s],
                              sem.at[s, 1]).wait()

    @pl.when(act_ref[b] == 1)
    def _compute():
        h = jnp.dot(x_ref[...], w1b[s], preferred_element_type=jnp.float32)
        h = h + b1_ref[0]
        h = 0.5 * h * (1.0 + lax.erf(h * 0.7071067811865476))
        y = jnp.dot(h, w2b[s], preferred_element_type=jnp.float32)
        y_ref[...] = y + b2_ref[0]

    @pl.when(act_ref[b] == 0)
    def _skip():
        y_ref[...] = jnp.zeros_like(y_ref)


def _ffn_call(be, act, fi, sl, nx, x_sorted, w1, b1, w2, b2):
    grid_spec = pltpu.PrefetchScalarGridSpec(
        num_scalar_prefetch=5,
        grid=(NB,),
        in_specs=[
            pl.BlockSpec((BT, D), lambda b, *_: (b, 0)),
            pl.BlockSpec(memory_space=pl.ANY),
            pl.BlockSpec((1, 1, F), lambda b, be, *_: (be[b], 0, 0)),
            pl.BlockSpec(memory_space=pl.ANY),
            pl.BlockSpec((1, 1, D), lambda b, be, *_: (be[b], 0, 0)),
        ],
        out_specs=pl.BlockSpec((BT, D), lambda b, *_: (b, 0)),
        scratch_shapes=[
            pltpu.VMEM((2, D, F), jnp.float32),
            pltpu.VMEM((2, F, D), jnp.float32),
            pltpu.SemaphoreType.DMA((2, 2)),
        ],
    )
    return pl.pallas_call(
        _ffn_body,
        grid_spec=grid_spec,
        out_shape=jax.ShapeDtypeStruct((NP, D), jnp.float32),
    )(be, act, fi, sl, nx, x_sorted,
      w1, b1.reshape(E, 1, F), w2, b2.reshape(E, 1, D))


def _combine_body(wa_ref, wb_ref, ye_ref, yo_ref, o_ref):
    o_ref[...] = wa_ref[...] * ye_ref[...] + wb_ref[...] * yo_ref[...]


def _combine_call(wa, wb, ye, yo):
    grid_spec = pl.GridSpec(
        grid=(T // BT,),
        in_specs=[
            pl.BlockSpec((BT, 1), lambda i: (i, 0)),
            pl.BlockSpec((BT, 1), lambda i: (i, 0)),
            pl.BlockSpec((BT, D), lambda i: (i, 0)),
            pl.BlockSpec((BT, D), lambda i: (i, 0)),
        ],
        out_specs=pl.BlockSpec((BT, D), lambda i: (i, 0)),
    )
    return pl.pallas_call(
        _combine_body,
        grid_spec=grid_spec,
        out_shape=jax.ShapeDtypeStruct((T, D), jnp.float32),
    )(wa, wb, ye, yo)


@jax.jit
def kernel(x, Wg, bg, W1, b1, W2, b2):
    orig_shape = x.shape
    xf = x.reshape(T, D)
    sc_scatter, sc_gather = _sc_kernels()
    de, do, wa, wb, be, act, fi, sl, nx = _router_call(xf, Wg, bg.reshape(1, E))
    x_sorted = sc_scatter(xf, de.reshape(T), do.reshape(T))
    y_sorted = _ffn_call(be.reshape(NB), act.reshape(NB), fi.reshape(NB),
                         sl.reshape(NB), nx.reshape(NB),
                         x_sorted, W1, b1, W2, b2)
    ye, yo = sc_gather(y_sorted, de.reshape(T), do.reshape(T))
    out = _combine_call(wa, wb, ye, yo)
    return out.reshape(orig_shape)


# speedup vs baseline: 1.5220x; 1.0338x over previous
import functools

import jax
import jax.numpy as jnp
from jax import lax
from jax.experimental import pallas as pl
from jax.experimental.pallas import tpu as pltpu
from jax.experimental.pallas import tpu_sc as plsc

D = 768
E = 8
F = 3072
T = 2048
K = 2
TK = T * K
BT = 256
NB = (TK + E * (BT - 1) + BT - 1) // BT
NP = NB * BT
NW = 32
TPW = T // NW


def _router_body(x_ref, wg_ref, bg_ref,
                 de_ref, do_ref, wa_ref, wb_ref, be_ref, act_ref,
                 fi_ref, sl_ref, nx_ref):
    x = x_ref[...]
    logits = jnp.dot(x, wg_ref[...], preferred_element_type=jnp.float32)
    logits = logits + bg_ref[...]

    eids = lax.broadcasted_iota(jnp.int32, (T, E), 1)
    l1 = jnp.max(logits, axis=1, keepdims=True)
    a1 = jnp.argmax(logits, axis=1).astype(jnp.int32)[:, None]
    masked = jnp.where(eids == a1, -jnp.inf, logits)
    l2 = jnp.max(masked, axis=1, keepdims=True)
    a2 = jnp.argmax(masked, axis=1).astype(jnp.int32)[:, None]
    wa = 1.0 / (1.0 + jnp.exp(l2 - l1))
    wb = 1.0 - wa

    e_slot = jnp.concatenate([a1, a2], axis=0)
    oh = (e_slot == lax.broadcasted_iota(jnp.int32, (TK, E), 1))
    oh = oh.astype(jnp.float32)

    nblk = TK // 128
    ohb = oh.reshape(nblk, 128, E)
    r = lax.broadcasted_iota(jnp.int32, (128, 128), 0)
    c = lax.broadcasted_iota(jnp.int32, (128, 128), 1)
    tri_incl = (r <= c).astype(jnp.float32)
    y = jnp.einsum('ale,lk->ake', ohb, tri_incl,
                   preferred_element_type=jnp.float32)
    totals = y[:, 127, :]
    rb = lax.broadcasted_iota(jnp.int32, (nblk, nblk), 0)
    cb = lax.broadcasted_iota(jnp.int32, (nblk, nblk), 1)
    tri_strict = (cb < rb).astype(jnp.float32)
    carry = jnp.einsum('ab,be->ae', tri_strict, totals,
                       preferred_element_type=jnp.float32)
    csum = (y + carry[:, None, :]).reshape(TK, E)

    counts = csum[TK - 1:TK, :]
    pc = jnp.ceil(counts / BT) * BT
    er = lax.broadcasted_iota(jnp.int32, (E, E), 0)
    ec = lax.broadcasted_iota(jnp.int32, (E, E), 1)
    tri_e = (er <= ec).astype(jnp.float32)
    cum_pc = jnp.einsum('ae,ef->af', pc, tri_e,
                        preferred_element_type=jnp.float32)
    po = cum_pc - pc

    rank = jnp.sum(oh * csum, axis=1, keepdims=True) - 1.0
    poe = jnp.sum(oh * po, axis=1, keepdims=True)
    dest = (poe + rank).astype(jnp.int32)
    de_ref[...] = dest[:T]
    do_ref[...] = dest[T:]
    ones16 = jnp.ones((1, 128), jnp.float32)
    wa_ref[...] = wa * ones16
    wb_ref[...] = wb * ones16

    s = lax.broadcasted_iota(jnp.int32, (NB, E), 0).astype(jnp.float32) * BT
    total = cum_pc[0, E - 1]
    be_raw = jnp.sum((s >= cum_pc).astype(jnp.int32), axis=1, keepdims=True)
    active = s[:, :1] < total
    eidx = lax.broadcasted_iota(jnp.int32, (1, E), 1)
    eb_last = jnp.max(jnp.where(pc > 0.0, eidx, -1), axis=1, keepdims=True)
    be = jnp.where(active, jnp.minimum(be_raw, E - 1), eb_last)
    be = be.astype(jnp.int32)
    be_ref[...] = be
    act_ref[...] = active.astype(jnp.int32)

    be_prev = jnp.concatenate(
        [jnp.full((1, 1), -1, jnp.int32), be[:-1]], axis=0)
    firstf = (be != be_prev).astype(jnp.float32)
    rnb = lax.broadcasted_iota(jnp.int32, (NB, NB), 0)
    cnb = lax.broadcasted_iota(jnp.int32, (NB, NB), 1)
    tri_nb = (cnb <= rnb).astype(jnp.float32)
    gidx = jnp.einsum('ab,bk->ak', tri_nb, firstf,
                      preferred_element_type=jnp.float32)
    slot = (gidx.astype(jnp.int32) - 1) % 2
    mask_next = jnp.logical_and(cnb > rnb, firstf[:, 0][None, :] > 0.0)
    pos = jnp.min(jnp.where(mask_next, cnb, 2 * NB), axis=1, keepdims=True)
    onehot_pos = (cnb == pos).astype(jnp.float32)
    nxt_f = jnp.einsum('ab,bk->ak', onehot_pos, be.astype(jnp.float32),
                       preferred_element_type=jnp.float32)
    nxt = jnp.where(pos < NB, nxt_f.astype(jnp.int32), -1)
    fi_ref[...] = firstf.astype(jnp.int32)
    sl_ref[...] = slot
    nx_ref[...] = nxt


def _router_call(xf, wg, bg2):
    outs = (
        jax.ShapeDtypeStruct((T, 1), jnp.int32),
        jax.ShapeDtypeStruct((T, 1), jnp.int32),
        jax.ShapeDtypeStruct((T, 128), jnp.float32),
        jax.ShapeDtypeStruct((T, 128), jnp.float32),
        jax.ShapeDtypeStruct((NB, 1), jnp.int32),
        jax.ShapeDtypeStruct((NB, 1), jnp.int32),
        jax.ShapeDtypeStruct((NB, 1), jnp.int32),
        jax.ShapeDtypeStruct((NB, 1), jnp.int32),
        jax.ShapeDtypeStruct((NB, 1), jnp.int32),
    )
    return pl.pallas_call(_router_body, out_shape=outs)(xf, wg, bg2)


@functools.lru_cache(maxsize=None)
def _sc_kernels():
    mesh = plsc.VectorSubcoreMesh(core_axis_name="c", subcore_axis_name="s",
                                  num_cores=2, num_subcores=16)

    @functools.partial(
        pl.kernel,
        out_type=(jax.ShapeDtypeStruct((NP, D), jnp.float32),
                  jax.ShapeDtypeStruct((NP, 128), jnp.float32)),
        mesh=mesh,
        scratch_types=[
            pltpu.VMEM((TPW, D), jnp.float32),
            pltpu.VMEM((TPW, 128), jnp.float32),
            pltpu.VMEM((TPW,), jnp.int32),
            pltpu.VMEM((TPW,), jnp.int32),
            pltpu.SemaphoreType.DMA,
            pltpu.SemaphoreType.DMA,
        ],
    )
    def _sc_scatter(x_hbm, de_hbm, do_hbm, wa_hbm, wb_hbm, out_hbm, ws_hbm,
                    rows_v, wrow_v, ide_v, ido_v, s1, s2):
        wid = lax.axis_index("s") * 2 + lax.axis_index("c")
        base = wid * TPW
        pltpu.sync_copy(x_hbm.at[pl.ds(base, TPW)], rows_v)
        pltpu.sync_copy(de_hbm.at[pl.ds(base, TPW)], ide_v)
        pltpu.sync_copy(do_hbm.at[pl.ds(base, TPW)], ido_v)
        pltpu.async_copy(rows_v, out_hbm.at[ide_v], s1).wait()
        pltpu.async_copy(rows_v, out_hbm.at[ido_v], s2).wait()
        pltpu.sync_copy(wa_hbm.at[pl.ds(base, TPW)], wrow_v)
        pltpu.async_copy(wrow_v, ws_hbm.at[ide_v], s1).wait()
        pltpu.sync_copy(wb_hbm.at[pl.ds(base, TPW)], wrow_v)
        pltpu.async_copy(wrow_v, ws_hbm.at[ido_v], s2).wait()

    @functools.partial(
        pl.kernel,
        out_type=jax.ShapeDtypeStruct((T, D), jnp.float32),
        mesh=mesh,
        scratch_types=[
            pltpu.VMEM((TPW, D), jnp.float32),
            pltpu.VMEM((TPW, D), jnp.float32),
            pltpu.VMEM((TPW,), jnp.int32),
            pltpu.VMEM((TPW,), jnp.int32),
            pltpu.SemaphoreType.DMA,
            pltpu.SemaphoreType.DMA,
        ],
    )
    def _sc_gather(ys_hbm, de_hbm, do_hbm, out_hbm,
                   rowsa_v, rowsb_v, ide_v, ido_v, s1, s2):
        wid = lax.axis_index("s") * 2 + lax.axis_index("c")
        base = wid * TPW
        pltpu.sync_copy(de_hbm.at[pl.ds(base, TPW)], ide_v)
        pltpu.sync_copy(do_hbm.at[pl.ds(base, TPW)], ido_v)
        ca = pltpu.async_copy(ys_hbm.at[ide_v], rowsa_v, s1)
        cb = pltpu.async_copy(ys_hbm.at[ido_v], rowsb_v, s2)
        ca.wait()
        cb.wait()

        def _add_row(r, carry):
            for c in range(D // 16):
                sl = pl.ds(c * 16, 16)
                rowsa_v[r, sl] = rowsa_v[r, sl] + rowsb_v[r, sl]
            return carry

        lax.fori_loop(0, TPW, _add_row, 0)
        pltpu.sync_copy(rowsa_v, out_hbm.at[pl.ds(base, TPW)])

    return _sc_scatter, _sc_gather


def _ffn_body(be_ref, act_ref, fi_ref, sl_ref, nx_ref,
              x_ref, w1_hbm, b1_ref, w2_hbm, b2_ref, ws_ref, y_ref,
              w1b, w2b, sem):
    b = pl.program_id(0)
    s = sl_ref[b]

    @pl.when(b == 0)
    def _prime():
        pltpu.make_async_copy(w1_hbm.at[be_ref[0]], w1b.at[0],
                              sem.at[0, 0]).start()
        pltpu.make_async_copy(w2_hbm.at[be_ref[0]], w2b.at[0],
                              sem.at[0, 1]).start()

    @pl.when(jnp.logical_and(fi_ref[b] == 1, nx_ref[b] >= 0))
    def _prefetch_next():
        nx = nx_ref[b]
        pltpu.make_async_copy(w1_hbm.at[nx], w1b.at[1 - s],
                              sem.at[1 - s, 0]).start()
        pltpu.make_async_copy(w2_hbm.at[nx], w2b.at[1 - s],
                              sem.at[1 - s, 1]).start()

    @pl.when(fi_ref[b] == 1)
    def _wait_cur():
        pltpu.make_async_copy(w1_hbm.at[be_ref[b]], w1b.at[s],
                              sem.at[s, 0]).wait()
        pltpu.make_async_copy(w2_hbm.at[be_ref[b]], w2b.at[s],
                              sem.at[s, 1]).wait()

    @pl.when(act_ref[b] == 1)
    def _compute():
        h = jnp.dot(x_ref[...], w1b[s], preferred_element_type=jnp.float32)
        h = h + b1_ref[0]
        h = 0.5 * h * (1.0 + lax.erf(h * 0.7071067811865476))
        y = jnp.dot(h, w2b[s], preferred_element_type=jnp.float32)
        y_ref[...] = (y + b2_ref[0]) * ws_ref[...][:, :1]

    @pl.when(act_ref[b] == 0)
    def _skip():
        y_ref[...] = jnp.zeros_like(y_ref)


def _ffn_call(be, act, fi, sl, nx, x_sorted, ws16, w1, b1, w2, b2):
    grid_spec = pltpu.PrefetchScalarGridSpec(
        num_scalar_prefetch=5,
        grid=(NB,),
        in_specs=[
            pl.BlockSpec((BT, D), lambda b, *_: (b, 0)),
            pl.BlockSpec(memory_space=pl.ANY),
            pl.BlockSpec((1, 1, F), lambda b, be, *_: (be[b], 0, 0)),
            pl.BlockSpec(memory_space=pl.ANY),
            pl.BlockSpec((1, 1, D), lambda b, be, *_: (be[b], 0, 0)),
            pl.BlockSpec((BT, 128), lambda b, *_: (b, 0)),
        ],
        out_specs=pl.BlockSpec((BT, D), lambda b, *_: (b, 0)),
        scratch_shapes=[
            pltpu.VMEM((2, D, F), jnp.float32),
            pltpu.VMEM((2, F, D), jnp.float32),
            pltpu.SemaphoreType.DMA((2, 2)),
        ],
    )
    return pl.pallas_call(
        _ffn_body,
        grid_spec=grid_spec,
        out_shape=jax.ShapeDtypeStruct((NP, D), jnp.float32),
    )(be, act, fi, sl, nx, x_sorted,
      w1, b1.reshape(E, 1, F), w2, b2.reshape(E, 1, D), ws16)


def _combine_body(wa_ref, wb_ref, ye_ref, yo_ref, o_ref):
    o_ref[...] = wa_ref[...] * ye_ref[...] + wb_ref[...] * yo_ref[...]


def _combine_call(wa, wb, ye, yo):
    grid_spec = pl.GridSpec(
        grid=(T // BT,),
        in_specs=[
            pl.BlockSpec((BT, 1), lambda i: (i, 0)),
            pl.BlockSpec((BT, 1), lambda i: (i, 0)),
            pl.BlockSpec((BT, D), lambda i: (i, 0)),
            pl.BlockSpec((BT, D), lambda i: (i, 0)),
        ],
        out_specs=pl.BlockSpec((BT, D), lambda i: (i, 0)),
    )
    return pl.pallas_call(
        _combine_body,
        grid_spec=grid_spec,
        out_shape=jax.ShapeDtypeStruct((T, D), jnp.float32),
    )(wa, wb, ye, yo)


@jax.jit
def kernel(x, Wg, bg, W1, b1, W2, b2):
    orig_shape = x.shape
    xf = x.reshape(T, D)
    sc_scatter, sc_gather = _sc_kernels()
    de, do, wa, wb, be, act, fi, sl, nx = _router_call(xf, Wg, bg.reshape(1, E))
    x_sorted, ws16 = sc_scatter(xf, de.reshape(T), do.reshape(T), wa, wb)
    y_sorted = _ffn_call(be.reshape(NB), act.reshape(NB), fi.reshape(NB),
                         sl.reshape(NB), nx.reshape(NB),
                         x_sorted, ws16, W1, b1, W2, b2)
    out = sc_gather(y_sorted, de.reshape(T), do.reshape(T))
    return out.reshape(orig_shape)


# speedup vs baseline: 1.5263x; 1.0028x over previous
import functools

import jax
import jax.numpy as jnp
from jax import lax
from jax.experimental import pallas as pl
from jax.experimental.pallas import tpu as pltpu
from jax.experimental.pallas import tpu_sc as plsc

D = 768
E = 8
F = 3072
T = 2048
K = 2
TK = T * K
BT = 256
NB = (TK + E * (BT - 1) + BT - 1) // BT
NP = NB * BT
NW = 32
TPW = T // NW


def _router_body(x_ref, wg_ref, bg_ref,
                 de_ref, do_ref, wa_ref, wb_ref, be_ref, act_ref,
                 fi_ref, sl_ref, nx_ref):
    x = x_ref[...]
    logits = jnp.dot(x, wg_ref[...], preferred_element_type=jnp.float32)
    logits = logits + bg_ref[...]

    eids = lax.broadcasted_iota(jnp.int32, (T, E), 1)
    l1 = jnp.max(logits, axis=1, keepdims=True)
    a1 = jnp.argmax(logits, axis=1).astype(jnp.int32)[:, None]
    masked = jnp.where(eids == a1, -jnp.inf, logits)
    l2 = jnp.max(masked, axis=1, keepdims=True)
    a2 = jnp.argmax(masked, axis=1).astype(jnp.int32)[:, None]
    wa = 1.0 / (1.0 + jnp.exp(l2 - l1))
    wb = 1.0 - wa

    e_slot = jnp.concatenate([a1, a2], axis=0)
    oh = (e_slot == lax.broadcasted_iota(jnp.int32, (TK, E), 1))
    oh = oh.astype(jnp.float32)

    nblk = TK // 128
    ohb = oh.reshape(nblk, 128, E)
    r = lax.broadcasted_iota(jnp.int32, (128, 128), 0)
    c = lax.broadcasted_iota(jnp.int32, (128, 128), 1)
    tri_incl = (r <= c).astype(jnp.float32)
    y = jnp.einsum('ale,lk->ake', ohb, tri_incl,
                   preferred_element_type=jnp.float32)
    totals = y[:, 127, :]
    rb = lax.broadcasted_iota(jnp.int32, (nblk, nblk), 0)
    cb = lax.broadcasted_iota(jnp.int32, (nblk, nblk), 1)
    tri_strict = (cb < rb).astype(jnp.float32)
    carry = jnp.einsum('ab,be->ae', tri_strict, totals,
                       preferred_element_type=jnp.float32)
    csum = (y + carry[:, None, :]).reshape(TK, E)

    counts = csum[TK - 1:TK, :]
    pc = jnp.ceil(counts / BT) * BT
    er = lax.broadcasted_iota(jnp.int32, (E, E), 0)
    ec = lax.broadcasted_iota(jnp.int32, (E, E), 1)
    tri_e = (er <= ec).astype(jnp.float32)
    cum_pc = jnp.einsum('ae,ef->af', pc, tri_e,
                        preferred_element_type=jnp.float32)
    po = cum_pc - pc

    rank = jnp.sum(oh * csum, axis=1, keepdims=True) - 1.0
    poe = jnp.sum(oh * po, axis=1, keepdims=True)
    dest = (poe + rank).astype(jnp.int32)
    de_ref[...] = dest[:T]
    do_ref[...] = dest[T:]
    ones16 = jnp.ones((1, 128), jnp.float32)
    wa_ref[...] = wa * ones16
    wb_ref[...] = wb * ones16

    s = lax.broadcasted_iota(jnp.int32, (NB, E), 0).astype(jnp.float32) * BT
    total = cum_pc[0, E - 1]
    be_raw = jnp.sum((s >= cum_pc).astype(jnp.int32), axis=1, keepdims=True)
    active = s[:, :1] < total
    eidx = lax.broadcasted_iota(jnp.int32, (1, E), 1)
    eb_last = jnp.max(jnp.where(pc > 0.0, eidx, -1), axis=1, keepdims=True)
    be = jnp.where(active, jnp.minimum(be_raw, E - 1), eb_last)
    be = be.astype(jnp.int32)
    be_ref[...] = be
    act_ref[...] = active.astype(jnp.int32)

    be_prev = jnp.concatenate(
        [jnp.full((1, 1), -1, jnp.int32), be[:-1]], axis=0)
    firstf = (be != be_prev).astype(jnp.float32)
    rnb = lax.broadcasted_iota(jnp.int32, (NB, NB), 0)
    cnb = lax.broadcasted_iota(jnp.int32, (NB, NB), 1)
    tri_nb = (cnb <= rnb).astype(jnp.float32)
    gidx = jnp.einsum('ab,bk->ak', tri_nb, firstf,
                      preferred_element_type=jnp.float32)
    slot = (gidx.astype(jnp.int32) - 1) % 2
    mask_next = jnp.logical_and(cnb > rnb, firstf[:, 0][None, :] > 0.0)
    pos = jnp.min(jnp.where(mask_next, cnb, 2 * NB), axis=1, keepdims=True)
    onehot_pos = (cnb == pos).astype(jnp.float32)
    nxt_f = jnp.einsum('ab,bk->ak', onehot_pos, be.astype(jnp.float32),
                       preferred_element_type=jnp.float32)
    nxt = jnp.where(pos < NB, nxt_f.astype(jnp.int32), -1)
    fi_ref[...] = firstf.astype(jnp.int32)
    sl_ref[...] = slot
    nx_ref[...] = nxt


def _router_call(xf, wg, bg2):
    outs = (
        jax.ShapeDtypeStruct((T, 1), jnp.int32),
        jax.ShapeDtypeStruct((T, 1), jnp.int32),
        jax.ShapeDtypeStruct((T, 128), jnp.float32),
        jax.ShapeDtypeStruct((T, 128), jnp.float32),
        jax.ShapeDtypeStruct((NB, 1), jnp.int32),
        jax.ShapeDtypeStruct((NB, 1), jnp.int32),
        jax.ShapeDtypeStruct((NB, 1), jnp.int32),
        jax.ShapeDtypeStruct((NB, 1), jnp.int32),
        jax.ShapeDtypeStruct((NB, 1), jnp.int32),
    )
    return pl.pallas_call(_router_body, out_shape=outs)(xf, wg, bg2)


@functools.lru_cache(maxsize=None)
def _sc_kernels():
    mesh = plsc.VectorSubcoreMesh(core_axis_name="c", subcore_axis_name="s",
                                  num_cores=2, num_subcores=16)

    @functools.partial(
        pl.kernel,
        out_type=(jax.ShapeDtypeStruct((NP, D), jnp.float32),
                  jax.ShapeDtypeStruct((NP, 128), jnp.float32)),
        mesh=mesh,
        scratch_types=[
            pltpu.VMEM((TPW, D), jnp.float32),
            pltpu.VMEM((TPW, 128), jnp.float32),
            pltpu.VMEM((TPW,), jnp.int32),
            pltpu.VMEM((TPW,), jnp.int32),
            pltpu.SemaphoreType.DMA,
            pltpu.SemaphoreType.DMA,
        ],
    )
    def _sc_scatter(x_hbm, de_hbm, do_hbm, wa_hbm, wb_hbm, out_hbm, ws_hbm,
                    rows_v, wrow_v, ide_v, ido_v, s1, s2):
        wid = lax.axis_index("s") * 2 + lax.axis_index("c")
        base = wid * TPW
        pltpu.sync_copy(x_hbm.at[pl.ds(base, TPW)], rows_v)
        pltpu.sync_copy(de_hbm.at[pl.ds(base, TPW)], ide_v)
        pltpu.sync_copy(do_hbm.at[pl.ds(base, TPW)], ido_v)
        pltpu.async_copy(rows_v, out_hbm.at[ide_v], s1).wait()
        pltpu.async_copy(rows_v, out_hbm.at[ido_v], s2).wait()
        pltpu.sync_copy(wa_hbm.at[pl.ds(base, TPW)], wrow_v)
        pltpu.async_copy(wrow_v, ws_hbm.at[ide_v], s1).wait()
        pltpu.sync_copy(wb_hbm.at[pl.ds(base, TPW)], wrow_v)
        pltpu.async_copy(wrow_v, ws_hbm.at[ido_v], s2).wait()

    @functools.partial(
        pl.kernel,
        out_type=jax.ShapeDtypeStruct((T, D), jnp.float32),
        mesh=mesh,
        scratch_types=[
            pltpu.VMEM((TPW, D), jnp.float32),
            pltpu.VMEM((TPW, D), jnp.float32),
            pltpu.VMEM((TPW,), jnp.int32),
            pltpu.VMEM((TPW,), jnp.int32),
            pltpu.SemaphoreType.DMA,
            pltpu.SemaphoreType.DMA,
        ],
    )
    def _sc_gather(ys_hbm, de_hbm, do_hbm, out_hbm,
                   rowsa_v, rowsb_v, ide_v, ido_v, s1, s2):
        wid = lax.axis_index("s") * 2 + lax.axis_index("c")
        base = wid * TPW
        pltpu.sync_copy(de_hbm.at[pl.ds(base, TPW)], ide_v)
        pltpu.sync_copy(do_hbm.at[pl.ds(base, TPW)], ido_v)
        ca = pltpu.async_copy(ys_hbm.at[ide_v], rowsa_v, s1)
        cb = pltpu.async_copy(ys_hbm.at[ido_v], rowsb_v, s2)
        ca.wait()
        cb.wait()

        def _add_row(r, carry):
            for c in range(D // 16):
                sl = pl.ds(c * 16, 16)
                rowsa_v[r, sl] = rowsa_v[r, sl] + rowsb_v[r, sl]
            return carry

        lax.fori_loop(0, TPW, _add_row, 0)
        pltpu.sync_copy(rowsa_v, out_hbm.at[pl.ds(base, TPW)])

    return _sc_scatter, _sc_gather


def _ffn_body(be_ref, act_ref, fi_ref, sl_ref, nx_ref,
              x_ref, w1_hbm, b1_ref, w2_hbm, b2_ref, ws_ref, y_ref,
              w1b, w2b, sem):
    b = pl.program_id(0)
    s = sl_ref[b]

    @pl.when(b == 0)
    def _prime():
        pltpu.make_async_copy(w1_hbm.at[be_ref[0]], w1b.at[0],
                              sem.at[0, 0]).start()
        pltpu.make_async_copy(w2_hbm.at[be_ref[0]], w2b.at[0],
                              sem.at[0, 1]).start()

    @pl.when(jnp.logical_and(fi_ref[b] == 1, nx_ref[b] >= 0))
    def _prefetch_next():
        nx = nx_ref[b]
        pltpu.make_async_copy(w1_hbm.at[nx], w1b.at[1 - s],
                              sem.at[1 - s, 0]).start()
        pltpu.make_async_copy(w2_hbm.at[nx], w2b.at[1 - s],
                              sem.at[1 - s, 1]).start()

    @pl.when(fi_ref[b] == 1)
    def _wait_cur():
        pltpu.make_async_copy(w1_hbm.at[be_ref[b]], w1b.at[s],
                              sem.at[s, 0]).wait()
        pltpu.make_async_copy(w2_hbm.at[be_ref[b]], w2b.at[s],
                              sem.at[s, 1]).wait()

    @pl.when(act_ref[b] == 1)
    def _compute():
        h = jnp.dot(x_ref[...], w1b[s], preferred_element_type=jnp.float32)
        h = h + b1_ref[0]
        h = 0.5 * h * (1.0 + lax.erf(h * 0.7071067811865476))
        y = jnp.dot(h, w2b[s], preferred_element_type=jnp.float32)
        y_ref[...] = (y + b2_ref[0]) * ws_ref[...][:, :1]

    @pl.when(act_ref[b] == 0)
    def _skip():
        y_ref[...] = jnp.zeros_like(y_ref)


def _ffn_call(be, act, fi, sl, nx, x_sorted, ws16, w1, b1, w2, b2):
    grid_spec = pltpu.PrefetchScalarGridSpec(
        num_scalar_prefetch=5,
        grid=(NB,),
        in_specs=[
            pl.BlockSpec((BT, D), lambda b, *_: (b, 0)),
            pl.BlockSpec(memory_space=pl.ANY),
            pl.BlockSpec((1, 1, F), lambda b, be, *_: (be[b], 0, 0)),
            pl.BlockSpec(memory_space=pl.ANY),
            pl.BlockSpec((1, 1, D), lambda b, be, *_: (be[b], 0, 0)),
            pl.BlockSpec((BT, 128), lambda b, *_: (b, 0)),
        ],
        out_specs=pl.BlockSpec((BT, D), lambda b, *_: (b, 0)),
        scratch_shapes=[
            pltpu.VMEM((2, D, F), jnp.float32),
            pltpu.VMEM((2, F, D), jnp.float32),
            pltpu.SemaphoreType.DMA((2, 2)),
        ],
    )
    return pl.pallas_call(
        _ffn_body,
        grid_spec=grid_spec,
        out_shape=jax.ShapeDtypeStruct((NP, D), jnp.float32),
    )(be, act, fi, sl, nx, x_sorted,
      w1, b1.reshape(E, 1, F), w2, b2.reshape(E, 1, D), ws16)


@jax.jit
def kernel(x, Wg, bg, W1, b1, W2, b2):
    orig_shape = x.shape
    xf = x.reshape(T, D)
    sc_scatter, sc_gather = _sc_kernels()
    de, do, wa, wb, be, act, fi, sl, nx = _router_call(xf, Wg, bg.reshape(1, E))
    x_sorted, ws16 = sc_scatter(xf, de.reshape(T), do.reshape(T), wa, wb)
    y_sorted = _ffn_call(be.reshape(NB), act.reshape(NB), fi.reshape(NB),
                         sl.reshape(NB), nx.reshape(NB),
                         x_sorted, ws16, W1, b1, W2, b2)
    out = sc_gather(y_sorted, de.reshape(T), do.reshape(T))
    return out.reshape(orig_shape)
